# Initial kernel scaffold; baseline (speedup 1.0000x reference)
import jax
import jax.numpy as jnp
from jax.experimental import pallas as pl


def kernel(ivectors, data):
    raise NotImplementedError("write your pallas kernel here")


# speedup vs baseline: 1.8738x; 1.8738x over previous
import functools

import jax
import jax.numpy as jnp
from jax import lax
from jax.experimental import pallas as pl
from jax.experimental.pallas import tpu as pltpu
from jax.experimental.pallas import tpu_sc as plsc

BATCH = 16384
HIST = 50
DIM = 64
B = BATCH * HIST
NC = 2
NS = 16
NW = NC * NS
BPW = B // NW
CH = 800
NCHUNK = BPW // CH
NP = NCHUNK // 2

_mesh = plsc.VectorSubcoreMesh(core_axis_name="c", subcore_axis_name="s")


@functools.partial(
    pl.kernel,
    out_type=jax.ShapeDtypeStruct((B, DIM), jnp.float32),
    mesh=_mesh,
    scratch_types=[
        pltpu.VMEM((CH,), jnp.int32),
        pltpu.VMEM((CH,), jnp.int32),
        pltpu.VMEM((CH, DIM), jnp.float32),
        pltpu.VMEM((CH, DIM), jnp.float32),
        pltpu.SemaphoreType.DMA,
        pltpu.SemaphoreType.DMA,
        pltpu.SemaphoreType.DMA,
        pltpu.SemaphoreType.DMA,
        pltpu.SemaphoreType.DMA,
        pltpu.SemaphoreType.DMA,
    ],
    compiler_params=pltpu.CompilerParams(use_tc_tiling_on_sc=False),
)
def _gather_kernel(table_hbm, idx_hbm, out_hbm,
                   idx0, idx1, rows0, rows1,
                   si0, si1, sg0, sg1, sw0, sw1):
    wid = lax.axis_index("s") * NC + lax.axis_index("c")
    base = wid * BPW
    idx_v = (idx0, idx1)
    rows_v = (rows0, rows1)
    s_i = (si0, si1)
    s_g = (sg0, sg1)
    s_w = (sw0, sw1)

    def step(j, b, wait_write, prefetch):
        off = base + j * CH
        pltpu.make_async_copy(idx_hbm.at[pl.ds(off, CH)], idx_v[b], s_i[b]).wait()
        if wait_write:
            pltpu.make_async_copy(rows_v[b], out_hbm.at[pl.ds(off, CH)], s_w[b]).wait()
        gather = pltpu.async_copy(table_hbm.at[idx_v[b]], rows_v[b], s_g[b])
        if prefetch:
            pltpu.async_copy(idx_hbm.at[pl.ds(off + CH, CH)], idx_v[1 - b], s_i[1 - b])
        gather.wait()
        pltpu.async_copy(rows_v[b], out_hbm.at[pl.ds(off, CH)], s_w[b])

    pltpu.async_copy(idx_hbm.at[pl.ds(base, CH)], idx0, si0)

    step(0, 0, wait_write=False, prefetch=True)
    step(1, 1, wait_write=False, prefetch=True)

    def body(k, carry):
        j = 2 * k
        step(j, 0, wait_write=True, prefetch=True)
        step(j + 1, 1, wait_write=True, prefetch=True)
        return carry

    lax.fori_loop(1, NP - 1, body, 0)

    step(NCHUNK - 2, 0, wait_write=True, prefetch=True)
    step(NCHUNK - 1, 1, wait_write=True, prefetch=False)

    pltpu.make_async_copy(rows0, out_hbm.at[pl.ds(base + (NCHUNK - 2) * CH, CH)], sw0).wait()
    pltpu.make_async_copy(rows1, out_hbm.at[pl.ds(base + (NCHUNK - 1) * CH, CH)], sw1).wait()


def kernel(ivectors, data):
    idx = data.reshape(-1).astype(jnp.int32)
    out = _gather_kernel(ivectors, idx)
    return out.reshape(BATCH, HIST, DIM)
